# baseline (speedup 1.0000x reference)
import functools

import jax
import jax.numpy as jnp
from jax.experimental import pallas as pl
from jax.experimental.pallas import tpu as pltpu


def _call(*args, **kwargs):
    return pl.pallas_call(*args, **kwargs)


def _rup(x, m):
    return ((x + m - 1) // m) * m


def _pad_last(a, to):
    if a.shape[-1] == to:
        return a
    pad = [(0, 0)] * (a.ndim - 1) + [(0, to - a.shape[-1])]
    return jnp.pad(a, pad)


def _fps_kern(npoint, N, NL, x_ref, xr_ref, idx_ref, nx_ref, dists):
    X = x_ref[0]
    Y = x_ref[1]
    Z = x_ref[2]
    iota2 = (jnp.remainder(
        jax.lax.broadcasted_iota(jnp.int32, (16, NL), 0), 8) * NL
        + jax.lax.broadcasted_iota(jnp.int32, (16, NL), 1))
    dists[...] = jnp.full((16, NL), 1e10, jnp.float32)

    def step(t, fars):
        far0, far1 = fars
        crow0 = xr_ref[0, pl.ds(far0, 1), :]
        crow1 = xr_ref[1, pl.ds(far1, 1), :]
        idx_ref[0, pl.ds(t, 1), :] = jnp.full((1, 128), far0, jnp.int32)
        idx_ref[1, pl.ds(t, 1), :] = jnp.full((1, 128), far1, jnp.int32)
        nx_ref[0, pl.ds(t, 1), :] = crow0
        nx_ref[1, pl.ds(t, 1), :] = crow1
        cx = jnp.concatenate([jnp.broadcast_to(crow0[0:1, 0:1], (8, 1)),
                              jnp.broadcast_to(crow1[0:1, 0:1], (8, 1))], 0)
        cy = jnp.concatenate([jnp.broadcast_to(crow0[0:1, 1:2], (8, 1)),
                              jnp.broadcast_to(crow1[0:1, 1:2], (8, 1))], 0)
        cz = jnp.concatenate([jnp.broadcast_to(crow0[0:1, 2:3], (8, 1)),
                              jnp.broadcast_to(crow1[0:1, 2:3], (8, 1))], 0)
        dx = X - cx
        dy = Y - cy
        dz = Z - cz
        d = (dx * dx + dy * dy) + dz * dz
        nd = jnp.minimum(dists[...], d)
        dists[...] = nd
        nd0 = nd[0:8]
        nd1 = nd[8:16]
        m0 = jnp.max(nd0)
        m1 = jnp.max(nd1)
        io8 = iota2[0:8]
        nf0 = jnp.min(jnp.where(nd0 == m0, io8, N)).astype(jnp.int32)
        nf1 = jnp.min(jnp.where(nd1 == m1, io8, N)).astype(jnp.int32)
        return (nf0, nf1)

    def step2(u, fars):
        return step(2 * u + 1, step(2 * u, fars))

    fars = jax.lax.fori_loop(0, npoint // 2, step2,
                             (jnp.int32(0), jnp.int32(0)))
    if npoint % 2:
        step(npoint - 1, fars)


def _fps(xyz, npoint):
    B, N, _ = xyz.shape
    NL = N // 8
    xr = xyz.transpose(2, 0, 1).reshape(3, B * 8, NL)
    xrows = _pad_last(xyz, 128)
    idx, nx = _call(
        functools.partial(_fps_kern, npoint, N, NL),
        out_shape=[jax.ShapeDtypeStruct((B, npoint, 128), jnp.int32),
                   jax.ShapeDtypeStruct((B, npoint, 128), jnp.float32)],
        scratch_shapes=[pltpu.VMEM((16, NL), jnp.float32)],
    )(xr, xrows)
    return idx[:, :, 0], nx[:, :, :3]


def _conv_kern(N, SB, r2, ns, nW1, has_fd, refs):
    if has_fd:
        (xyzT_ref, q_ref, Fhi_ref, W0_ref, b0_ref, W1_ref, b1_ref,
         fd_ref, W2a_ref, W2b_ref, b2_ref, o_ref) = refs
    else:
        (xyzT_ref, q_ref, Fhi_ref, W0_ref, b0_ref, W1_ref, b1_ref,
         o_ref) = refs
    q = q_ref[0]
    xx = xyzT_ref[0, 0:1, :]
    xy = xyzT_ref[0, 1:2, :]
    xz = xyzT_ref[0, 2:3, :]
    dx = q[:, 0:1] - xx
    dy = q[:, 1:2] - xy
    dz = q[:, 2:3] - xz
    sq = (dx * dx + dy * dy) + dz * dz
    ioN = jax.lax.broadcasted_iota(jnp.int32, (SB, N), 1)
    mask = sq <= r2
    rank = mask.astype(jnp.int32)
    k = 1
    while k < N:
        rolled = pltpu.roll(rank, k, axis=1)
        rank = rank + jnp.where(ioN >= k, rolled, 0)
        k *= 2
    cnt = rank[:, N - 1:N]
    Fhi = Fhi_ref[0]
    W0 = W0_ref[...]
    b0 = b0_ref[...]
    W1 = W1_ref[...]
    b1 = b1_ref[...]
    acc = None
    for j in range(ns):
        selj = jnp.where(cnt >= j + 1, j + 1, 1)
        ohb = mask & (rank == selj)
        if has_fd:
            ohb = ohb | ((cnt == 0) & (ioN == N - 1))
        ohj = ohb.astype(jnp.float32)
        g = jax.lax.dot_general(ohj, Fhi, (((1,), (0,)), ((), ())),
                                preferred_element_type=jnp.float32)
        g = g - q
        h = jnp.maximum(
            jax.lax.dot_general(g, W0, (((1,), (0,)), ((), ())),
                                preferred_element_type=jnp.float32) + b0, 0.0)
        if nW1:
            h = jnp.maximum(
                jax.lax.dot_general(h, W1, (((1,), (0,)), ((), ())),
                                    preferred_element_type=jnp.float32) + b1,
                0.0)
        acc = h if acc is None else jnp.maximum(acc, h)
    if has_fd:
        fd = fd_ref[0]
        o = jnp.maximum(
            jax.lax.dot_general(acc, W2a_ref[...], (((1,), (0,)), ((), ())),
                                preferred_element_type=jnp.float32)
            + jax.lax.dot_general(fd, W2b_ref[...], (((1,), (0,)), ((), ())),
                                  preferred_element_type=jnp.float32)
            + b2_ref[...], 0.0)
    else:
        o = acc
    o_ref[0] = o


def _conv_level(xyz, new_xyz, feat, radius, nsample, Ws, bs, SB,
                feat_d=None, W2=None, b2=None):
    B, N, _ = xyz.shape
    S = new_xyz.shape[1]
    Cf = feat.shape[-1]
    Cin = 3 + Cf
    CP = _rup(Cin, 128)
    H1 = Ws[0].shape[1]
    H1P = _rup(H1, 128)
    nW1 = len(Ws) > 1
    if nW1:
        H2 = Ws[1].shape[1]
        H2P = _rup(H2, 128)
    else:
        H2, H2P = H1, H1P
    has_fd = feat_d is not None

    xyzT = _pad_last(xyz, 8).transpose(0, 2, 1)
    q_pad = _pad_last(new_xyz, CP)
    F_all = _pad_last(jnp.concatenate([xyz, feat], -1), CP)
    W0p = _pad_last(jnp.pad(Ws[0], ((0, CP - Cin), (0, 0))), H1P)
    b0p = _pad_last(bs[0][None, :], H1P)
    if nW1:
        W1p = _pad_last(jnp.pad(Ws[1], ((0, H1P - H1), (0, 0))), H2P)
        b1p = _pad_last(bs[1][None, :], H2P)
    else:
        W1p = jnp.zeros((8, 128), jnp.float32)
        b1p = jnp.zeros((1, 128), jnp.float32)

    r2 = float(radius) * float(radius)
    nsb = S // SB
    grid = (B, nsb)
    in_specs = [
        pl.BlockSpec((1, 8, N), lambda b, s: (b, 0, 0)),
        pl.BlockSpec((1, SB, CP), lambda b, s: (b, s, 0)),
        pl.BlockSpec((1, N, CP), lambda b, s: (b, 0, 0)),
        pl.BlockSpec(W0p.shape, lambda b, s: (0, 0)),
        pl.BlockSpec(b0p.shape, lambda b, s: (0, 0)),
        pl.BlockSpec(W1p.shape, lambda b, s: (0, 0)),
        pl.BlockSpec(b1p.shape, lambda b, s: (0, 0)),
    ]
    args = [xyzT, q_pad, F_all, W0p, b0p, W1p, b1p]
    if has_fd:
        Cd = feat_d.shape[-1]
        CdP = _rup(Cd, 128)
        H3 = W2.shape[1]
        H3P = _rup(H3, 128)
        fdp = _pad_last(feat_d, CdP)
        W2a = _pad_last(jnp.pad(W2[:H2], ((0, H2P - H2), (0, 0))), H3P)
        W2b = _pad_last(jnp.pad(W2[H2:], ((0, CdP - Cd), (0, 0))), H3P)
        b2p = _pad_last(b2[None, :], H3P)
        in_specs += [
            pl.BlockSpec((1, SB, CdP), lambda b, s: (b, s, 0)),
            pl.BlockSpec(W2a.shape, lambda b, s: (0, 0)),
            pl.BlockSpec(W2b.shape, lambda b, s: (0, 0)),
            pl.BlockSpec(b2p.shape, lambda b, s: (0, 0)),
        ]
        args += [fdp, W2a, W2b, b2p]
        HoutP = H3P
    else:
        HoutP = H2P

    def kern(*refs):
        _conv_kern(N, SB, r2, nsample, nW1, has_fd, refs)

    out = _call(
        kern,
        grid=grid,
        in_specs=in_specs,
        out_specs=pl.BlockSpec((1, SB, HoutP), lambda b, s: (b, s, 0)),
        out_shape=jax.ShapeDtypeStruct((B, S, HoutP), jnp.float32),
    )(*args)
    return out


def _down0_kern(x_ref, W_ref, b_ref, o_ref):
    o_ref[0] = jnp.maximum(
        jax.lax.dot_general(x_ref[0], W_ref[...], (((1,), (0,)), ((), ())),
                            preferred_element_type=jnp.float32) + b_ref[...],
        0.0)


def _down0(feat, W, b):
    B, N, C = feat.shape
    H = W.shape[1]
    HP = _rup(H, 128)
    xp = _pad_last(feat, 8)
    Wp = _pad_last(jnp.pad(W, ((0, 8 - C), (0, 0))), HP)
    bp = _pad_last(b[None, :], HP)
    out = _call(
        _down0_kern,
        grid=(B,),
        in_specs=[pl.BlockSpec((1, N, 8), lambda bb: (bb, 0, 0)),
                  pl.BlockSpec(Wp.shape, lambda bb: (0, 0)),
                  pl.BlockSpec(bp.shape, lambda bb: (0, 0))],
        out_specs=pl.BlockSpec((1, N, HP), lambda bb: (bb, 0, 0)),
        out_shape=jax.ShapeDtypeStruct((B, N, HP), jnp.float32),
    )(xp, Wp, bp)
    return out


def kernel(pc, feat, down0_W, down0_b, d1_W0, d1_b0, d1_W1, d1_b1,
           d2_W0, d2_b0, d2_W1, d2_b1, d3_W0, d3_b0, d3_W1, d3_b1,
           d4_W0, d4_b0, d4_W1, d4_b1, u4_W1, u4_b1, u4_W2, u4_b2,
           u3_W1, u3_b1, u3_W2, u3_b2, u2_W1, u2_b1, u2_W2, u2_b2,
           u1_W1, u1_b1, u1_W2, u1_b2):
    f0p = _down0(feat, down0_W, down0_b)
    f0 = f0p[:, :, :32]

    i1, x1 = _fps(pc, 2048)
    f1 = _conv_level(pc, x1, f0, 0.5, 16, [d1_W0, d1_W1], [d1_b0, d1_b1],
                     SB=256)[:, :, :64]
    i2, x2 = _fps(x1, 512)
    f2 = _conv_level(x1, x2, f1, 1.0, 16, [d2_W0, d2_W1], [d2_b0, d2_b1],
                     SB=512)[:, :, :128]
    i3, x3 = _fps(x2, 128)
    f3 = _conv_level(x2, x3, f2, 2.0, 16, [d3_W0, d3_W1], [d3_b0, d3_b1],
                     SB=128)[:, :, :192]
    i4, x4 = _fps(x3, 64)
    f4 = _conv_level(x3, x4, f3, 4.0, 16, [d4_W0, d4_W1], [d4_b0, d4_b1],
                     SB=64)[:, :, :192]

    u3 = _conv_level(x4, x3, f4, 6.0, 8, [u4_W1], [u4_b1], SB=128,
                     feat_d=f3, W2=u4_W2, b2=u4_b2)[:, :, :192]
    u2 = _conv_level(x3, x2, u3, 3.0, 8, [u3_W1], [u3_b1], SB=512,
                     feat_d=f2, W2=u3_W2, b2=u3_b2)[:, :, :128]
    u1 = _conv_level(x2, x1, u2, 1.5, 8, [u2_W1], [u2_b1], SB=512,
                     feat_d=f1, W2=u2_W2, b2=u2_b2)[:, :, :64]
    u0 = _conv_level(x1, pc, u1, 0.75, 8, [u1_W1], [u1_b1], SB=1024,
                     feat_d=f0, W2=u1_W2, b2=u1_b2)[:, :, :32]

    return (x1, x2, x3, x4), (i1, i2, i3, i4), (u0, u1, u2, u3)


# speedup vs baseline: 1.1166x; 1.1166x over previous
import functools

import jax
import jax.numpy as jnp
from jax.experimental import pallas as pl
from jax.experimental.pallas import tpu as pltpu


def _call(*args, **kwargs):
    return pl.pallas_call(*args, **kwargs)


def _rup(x, m):
    return ((x + m - 1) // m) * m


def _pad_last(a, to):
    if a.shape[-1] == to:
        return a
    pad = [(0, 0)] * (a.ndim - 1) + [(0, to - a.shape[-1])]
    return jnp.pad(a, pad)


def _fps_kern(npoint, N, NL, x_ref, xr_ref, idx_ref, nx_ref, dists):
    X = x_ref[0]
    Y = x_ref[1]
    Z = x_ref[2]
    iota2 = (jnp.remainder(
        jax.lax.broadcasted_iota(jnp.int32, (16, NL), 0), 8) * NL
        + jax.lax.broadcasted_iota(jnp.int32, (16, NL), 1))
    dists[...] = jnp.full((16, NL), 1e10, jnp.float32)

    def step(t, fars):
        far0, far1 = fars
        crow0 = xr_ref[0, pl.ds(far0, 1), :]
        crow1 = xr_ref[1, pl.ds(far1, 1), :]
        idx_ref[0, pl.ds(t, 1), :] = jnp.full((1, 128), far0, jnp.int32)
        idx_ref[1, pl.ds(t, 1), :] = jnp.full((1, 128), far1, jnp.int32)
        nx_ref[0, pl.ds(t, 1), :] = crow0
        nx_ref[1, pl.ds(t, 1), :] = crow1
        cx = jnp.concatenate([jnp.broadcast_to(crow0[0:1, 0:1], (8, 1)),
                              jnp.broadcast_to(crow1[0:1, 0:1], (8, 1))], 0)
        cy = jnp.concatenate([jnp.broadcast_to(crow0[0:1, 1:2], (8, 1)),
                              jnp.broadcast_to(crow1[0:1, 1:2], (8, 1))], 0)
        cz = jnp.concatenate([jnp.broadcast_to(crow0[0:1, 2:3], (8, 1)),
                              jnp.broadcast_to(crow1[0:1, 2:3], (8, 1))], 0)
        dx = X - cx
        dy = Y - cy
        dz = Z - cz
        d = (dx * dx + dy * dy) + dz * dz
        nd = jnp.minimum(dists[...], d)
        dists[...] = nd
        nd0 = nd[0:8]
        nd1 = nd[8:16]
        m0 = jnp.max(nd0)
        m1 = jnp.max(nd1)
        io8 = iota2[0:8]
        nf0 = jnp.min(jnp.where(nd0 == m0, io8, N)).astype(jnp.int32)
        nf1 = jnp.min(jnp.where(nd1 == m1, io8, N)).astype(jnp.int32)
        return (nf0, nf1)

    def step2(u, fars):
        return step(2 * u + 1, step(2 * u, fars))

    fars = jax.lax.fori_loop(0, npoint // 2, step2,
                             (jnp.int32(0), jnp.int32(0)))
    if npoint % 2:
        step(npoint - 1, fars)


def _fps(xyz, npoint):
    B, N, _ = xyz.shape
    NL = N // 8
    xr = xyz.transpose(2, 0, 1).reshape(3, B * 8, NL)
    xrows = _pad_last(xyz, 128)
    idx, nx = _call(
        functools.partial(_fps_kern, npoint, N, NL),
        out_shape=[jax.ShapeDtypeStruct((B, npoint, 128), jnp.int32),
                   jax.ShapeDtypeStruct((B, npoint, 128), jnp.float32)],
        scratch_shapes=[pltpu.VMEM((16, NL), jnp.float32)],
    )(xr, xrows)
    return idx[:, :, 0], nx[:, :, :3]


def _conv_kern(N, SB, r2, ns, nW1, has_fd, refs):
    if has_fd:
        (xyzT_ref, q_ref, Fhi_ref, W0_ref, b0_ref, W1_ref, b1_ref,
         fd_ref, W2a_ref, W2b_ref, b2_ref, o_ref) = refs
    else:
        (xyzT_ref, q_ref, Fhi_ref, W0_ref, b0_ref, W1_ref, b1_ref,
         o_ref) = refs
    q = q_ref[0]
    xx = xyzT_ref[0, 0:1, :]
    xy = xyzT_ref[0, 1:2, :]
    xz = xyzT_ref[0, 2:3, :]
    dx = q[:, 0:1] - xx
    dy = q[:, 1:2] - xy
    dz = q[:, 2:3] - xz
    sq = (dx * dx + dy * dy) + dz * dz
    ioN = jax.lax.broadcasted_iota(jnp.int32, (SB, N), 1)
    mask = sq <= r2
    rank = mask.astype(jnp.int32)
    k = 1
    while k < N:
        rolled = pltpu.roll(rank, k, axis=1)
        rank = rank + jnp.where(ioN >= k, rolled, 0)
        k *= 2
    cnt = rank[:, N - 1:N]
    Fhi = Fhi_ref[0]
    W0 = W0_ref[...]
    b0 = b0_ref[...]
    W1 = W1_ref[...]
    b1 = b1_ref[...]
    acc = None
    for j in range(ns):
        selj = jnp.where(cnt >= j + 1, j + 1, 1)
        ohb = mask & (rank == selj)
        if has_fd:
            ohb = ohb | ((cnt == 0) & (ioN == N - 1))
        ohj = ohb.astype(jnp.float32)
        g = jax.lax.dot_general(ohj, Fhi, (((1,), (0,)), ((), ())),
                                preferred_element_type=jnp.float32)
        g = g - q
        h = jnp.maximum(
            jax.lax.dot_general(g, W0, (((1,), (0,)), ((), ())),
                                preferred_element_type=jnp.float32) + b0, 0.0)
        if nW1:
            h = jnp.maximum(
                jax.lax.dot_general(h, W1, (((1,), (0,)), ((), ())),
                                    preferred_element_type=jnp.float32) + b1,
                0.0)
        acc = h if acc is None else jnp.maximum(acc, h)
    if has_fd:
        fd = fd_ref[0]
        o = jnp.maximum(
            jax.lax.dot_general(acc, W2a_ref[...], (((1,), (0,)), ((), ())),
                                preferred_element_type=jnp.float32)
            + jax.lax.dot_general(fd, W2b_ref[...], (((1,), (0,)), ((), ())),
                                  preferred_element_type=jnp.float32)
            + b2_ref[...], 0.0)
    else:
        o = acc
    o_ref[0] = o


def _conv_level(xyz, new_xyz, feat, radius, nsample, Ws, bs, SB,
                feat_d=None, W2=None, b2=None):
    B, N, _ = xyz.shape
    S = new_xyz.shape[1]
    Cf = feat.shape[-1]
    Cin = 3 + Cf
    CP = _rup(Cin, 128)
    H1 = Ws[0].shape[1]
    H1P = _rup(H1, 128)
    nW1 = len(Ws) > 1
    if nW1:
        H2 = Ws[1].shape[1]
        H2P = _rup(H2, 128)
    else:
        H2, H2P = H1, H1P
    has_fd = feat_d is not None

    xyzT = _pad_last(xyz, 8).transpose(0, 2, 1)
    q_pad = _pad_last(new_xyz, CP)
    F_all = _pad_last(jnp.concatenate([xyz, feat], -1), CP)
    W0p = _pad_last(jnp.pad(Ws[0], ((0, CP - Cin), (0, 0))), H1P)
    b0p = _pad_last(bs[0][None, :], H1P)
    if nW1:
        W1p = _pad_last(jnp.pad(Ws[1], ((0, H1P - H1), (0, 0))), H2P)
        b1p = _pad_last(bs[1][None, :], H2P)
    else:
        W1p = jnp.zeros((8, 128), jnp.float32)
        b1p = jnp.zeros((1, 128), jnp.float32)

    r2 = float(radius) * float(radius)
    nsb = S // SB
    grid = (B, nsb)
    in_specs = [
        pl.BlockSpec((1, 8, N), lambda b, s: (b, 0, 0)),
        pl.BlockSpec((1, SB, CP), lambda b, s: (b, s, 0)),
        pl.BlockSpec((1, N, CP), lambda b, s: (b, 0, 0)),
        pl.BlockSpec(W0p.shape, lambda b, s: (0, 0)),
        pl.BlockSpec(b0p.shape, lambda b, s: (0, 0)),
        pl.BlockSpec(W1p.shape, lambda b, s: (0, 0)),
        pl.BlockSpec(b1p.shape, lambda b, s: (0, 0)),
    ]
    args = [xyzT, q_pad, F_all, W0p, b0p, W1p, b1p]
    if has_fd:
        Cd = feat_d.shape[-1]
        CdP = _rup(Cd, 128)
        H3 = W2.shape[1]
        H3P = _rup(H3, 128)
        fdp = _pad_last(feat_d, CdP)
        W2a = _pad_last(jnp.pad(W2[:H2], ((0, H2P - H2), (0, 0))), H3P)
        W2b = _pad_last(jnp.pad(W2[H2:], ((0, CdP - Cd), (0, 0))), H3P)
        b2p = _pad_last(b2[None, :], H3P)
        in_specs += [
            pl.BlockSpec((1, SB, CdP), lambda b, s: (b, s, 0)),
            pl.BlockSpec(W2a.shape, lambda b, s: (0, 0)),
            pl.BlockSpec(W2b.shape, lambda b, s: (0, 0)),
            pl.BlockSpec(b2p.shape, lambda b, s: (0, 0)),
        ]
        args += [fdp, W2a, W2b, b2p]
        HoutP = H3P
    else:
        HoutP = H2P

    def kern(*refs):
        _conv_kern(N, SB, r2, nsample, nW1, has_fd, refs)

    out = _call(
        kern,
        grid=grid,
        in_specs=in_specs,
        out_specs=pl.BlockSpec((1, SB, HoutP), lambda b, s: (b, s, 0)),
        out_shape=jax.ShapeDtypeStruct((B, S, HoutP), jnp.float32),
    )(*args)
    return out


def _down0_kern(x_ref, W_ref, b_ref, o_ref):
    o_ref[0] = jnp.maximum(
        jax.lax.dot_general(x_ref[0], W_ref[...], (((1,), (0,)), ((), ())),
                            preferred_element_type=jnp.float32) + b_ref[...],
        0.0)


def _down0(feat, W, b):
    B, N, C = feat.shape
    H = W.shape[1]
    HP = _rup(H, 128)
    xp = _pad_last(feat, 8)
    Wp = _pad_last(jnp.pad(W, ((0, 8 - C), (0, 0))), HP)
    bp = _pad_last(b[None, :], HP)
    out = _call(
        _down0_kern,
        grid=(B,),
        in_specs=[pl.BlockSpec((1, N, 8), lambda bb: (bb, 0, 0)),
                  pl.BlockSpec(Wp.shape, lambda bb: (0, 0)),
                  pl.BlockSpec(bp.shape, lambda bb: (0, 0))],
        out_specs=pl.BlockSpec((1, N, HP), lambda bb: (bb, 0, 0)),
        out_shape=jax.ShapeDtypeStruct((B, N, HP), jnp.float32),
    )(xp, Wp, bp)
    return out


def kernel(pc, feat, down0_W, down0_b, d1_W0, d1_b0, d1_W1, d1_b1,
           d2_W0, d2_b0, d2_W1, d2_b1, d3_W0, d3_b0, d3_W1, d3_b1,
           d4_W0, d4_b0, d4_W1, d4_b1, u4_W1, u4_b1, u4_W2, u4_b2,
           u3_W1, u3_b1, u3_W2, u3_b2, u2_W1, u2_b1, u2_W2, u2_b2,
           u1_W1, u1_b1, u1_W2, u1_b2):
    f0p = _down0(feat, down0_W, down0_b)
    f0 = f0p[:, :, :32]

    i1, x1 = _fps(pc, 2048)
    f1 = _conv_level(pc, x1, f0, 0.5, 16, [d1_W0, d1_W1], [d1_b0, d1_b1],
                     SB=128)[:, :, :64]
    i2, x2 = _fps(x1, 512)
    f2 = _conv_level(x1, x2, f1, 1.0, 16, [d2_W0, d2_W1], [d2_b0, d2_b1],
                     SB=512)[:, :, :128]
    i3, x3 = _fps(x2, 128)
    f3 = _conv_level(x2, x3, f2, 2.0, 16, [d3_W0, d3_W1], [d3_b0, d3_b1],
                     SB=128)[:, :, :192]
    i4, x4 = _fps(x3, 64)
    f4 = _conv_level(x3, x4, f3, 4.0, 16, [d4_W0, d4_W1], [d4_b0, d4_b1],
                     SB=64)[:, :, :192]

    u3 = _conv_level(x4, x3, f4, 6.0, 8, [u4_W1], [u4_b1], SB=128,
                     feat_d=f3, W2=u4_W2, b2=u4_b2)[:, :, :192]
    u2 = _conv_level(x3, x2, u3, 3.0, 8, [u3_W1], [u3_b1], SB=512,
                     feat_d=f2, W2=u3_W2, b2=u3_b2)[:, :, :128]
    u1 = _conv_level(x2, x1, u2, 1.5, 8, [u2_W1], [u2_b1], SB=512,
                     feat_d=f1, W2=u2_W2, b2=u2_b2)[:, :, :64]
    u0 = _conv_level(x1, pc, u1, 0.75, 8, [u1_W1], [u1_b1], SB=512,
                     feat_d=f0, W2=u1_W2, b2=u1_b2)[:, :, :32]

    return (x1, x2, x3, x4), (i1, i2, i3, i4), (u0, u1, u2, u3)
